# Initial kernel scaffold; baseline (speedup 1.0000x reference)
import jax
import jax.numpy as jnp
from jax.experimental import pallas as pl


def kernel(tokens, embedding_weight):
    raise NotImplementedError("write your pallas kernel here")


# speedup vs baseline: 1.1112x; 1.1112x over previous
import functools

import jax
import jax.numpy as jnp
from jax import lax
from jax.experimental import pallas as pl
from jax.experimental.pallas import tpu as pltpu
from jax.experimental.pallas import tpu_sc as plsc

VOCAB_SIZE = 1000000
EMB_SIZE = 32
BATCH = 16384
HIST_LEN = 50

B_FLAT = BATCH * HIST_LEN


def _make_gather(V, D, B, chunk):
    info = plsc.get_sparse_core_info()
    NC, NS = info.num_cores, info.num_subcores
    NW = NC * NS
    b_per_w = B // NW
    assert b_per_w % chunk == 0 and chunk % 8 == 0
    n_chunks = b_per_w // chunk
    mesh = plsc.VectorSubcoreMesh(core_axis_name="c", subcore_axis_name="s")

    @functools.partial(
        pl.kernel,
        mesh=mesh,
        out_type=jax.ShapeDtypeStruct((B, D), jnp.float32),
        scratch_types=[
            pltpu.VMEM((chunk,), jnp.int32),
            pltpu.VMEM((chunk, D), jnp.float32),
            pltpu.SemaphoreType.DMA,
        ],
        compiler_params=pltpu.CompilerParams(use_tc_tiling_on_sc=False),
    )
    def k(tok_hbm, table_hbm, out_hbm, idx_v, rows_v, sem):
        wid = lax.axis_index("s") * NC + lax.axis_index("c")
        w_base = wid * b_per_w

        def body(i, carry):
            base = w_base + i * chunk
            pltpu.sync_copy(tok_hbm.at[pl.ds(base, chunk)], idx_v)
            pltpu.async_copy(table_hbm.at[idx_v], rows_v, sem).wait()
            pltpu.sync_copy(rows_v, out_hbm.at[pl.ds(base, chunk)])
            return carry

        lax.fori_loop(0, n_chunks, body, 0)

    return k


_gather = _make_gather(VOCAB_SIZE, EMB_SIZE, B_FLAT, chunk=3200)


def kernel(tokens, embedding_weight):
    flat = tokens.reshape(-1).astype(jnp.int32)
    out = _gather(flat, embedding_weight)
    return out.reshape(tokens.shape + (EMB_SIZE,))


# speedup vs baseline: 1.1139x; 1.0025x over previous
import functools

import jax
import jax.numpy as jnp
from jax import lax
from jax.experimental import pallas as pl
from jax.experimental.pallas import tpu as pltpu
from jax.experimental.pallas import tpu_sc as plsc

VOCAB_SIZE = 1000000
EMB_SIZE = 32
BATCH = 16384
HIST_LEN = 50

B_FLAT = BATCH * HIST_LEN


def _make_gather(V, D, B, chunk):
    info = plsc.get_sparse_core_info()
    NC, NS = info.num_cores, info.num_subcores
    NW = NC * NS
    b_per_w = B // NW
    assert b_per_w % chunk == 0 and chunk % 8 == 0
    n_chunks = b_per_w // chunk
    assert n_chunks % 2 == 0 and n_chunks >= 4
    mesh = plsc.VectorSubcoreMesh(core_axis_name="c", subcore_axis_name="s")

    @functools.partial(
        pl.kernel,
        mesh=mesh,
        out_type=jax.ShapeDtypeStruct((B, D), jnp.float32),
        scratch_types=[
            pltpu.VMEM((b_per_w,), jnp.int32),
            pltpu.VMEM((2, chunk, D), jnp.float32),
            pltpu.SemaphoreType.DMA,
            pltpu.SemaphoreType.DMA,
            pltpu.SemaphoreType.DMA,
            pltpu.SemaphoreType.DMA,
        ],
        compiler_params=pltpu.CompilerParams(use_tc_tiling_on_sc=False),
    )
    def k(tok_hbm, table_hbm, out_hbm, idx_v, rows_v, sg0, sg1, so0, so1):
        wid = lax.axis_index("s") * NC + lax.axis_index("c")
        w_base = wid * b_per_w
        sg = (sg0, sg1)
        so = (so0, so1)

        def gather(g, b):
            return pltpu.make_async_copy(
                table_hbm.at[idx_v.at[pl.ds(g * chunk, chunk)]],
                rows_v.at[b], sg[b])

        def store(g, b):
            return pltpu.make_async_copy(
                rows_v.at[b], out_hbm.at[pl.ds(w_base + g * chunk, chunk)],
                so[b])

        pltpu.sync_copy(tok_hbm.at[pl.ds(w_base, b_per_w)], idx_v)

        gather(0, 0).start()
        gather(1, 1).start()

        def body(o, carry):
            for b in (0, 1):
                g = 2 * o + b
                gather(g - 2, b).wait()
                store(g - 2, b).start()
                store(g - 2, b).wait()
                gather(g, b).start()
            return carry

        lax.fori_loop(1, n_chunks // 2, body, 0)

        for b in (0, 1):
            g = n_chunks - 2 + b
            gather(g, b).wait()
            store(g, b).start()
            store(g, b).wait()

    return k


_gather = _make_gather(VOCAB_SIZE, EMB_SIZE, B_FLAT, chunk=1280)


def kernel(tokens, embedding_weight):
    flat = tokens.reshape(-1).astype(jnp.int32)
    out = _gather(flat, embedding_weight)
    return out.reshape(tokens.shape + (EMB_SIZE,))


# speedup vs baseline: 1.8128x; 1.6274x over previous
import functools

import jax
import jax.numpy as jnp
from jax import lax
from jax.experimental import pallas as pl
from jax.experimental.pallas import tpu as pltpu
from jax.experimental.pallas import tpu_sc as plsc

VOCAB_SIZE = 1000000
EMB_SIZE = 32
BATCH = 16384
HIST_LEN = 50

B_FLAT = BATCH * HIST_LEN


def _make_gather(V, D, B, chunk):
    info = plsc.get_sparse_core_info()
    NC, NS = info.num_cores, info.num_subcores
    NW = NC * NS
    b_per_w = B // NW
    assert b_per_w % chunk == 0 and chunk % 8 == 0
    n_chunks = b_per_w // chunk
    assert n_chunks % 2 == 0 and n_chunks >= 4
    mesh = plsc.VectorSubcoreMesh(core_axis_name="c", subcore_axis_name="s")

    @functools.partial(
        pl.kernel,
        mesh=mesh,
        out_type=jax.ShapeDtypeStruct((B, D), jnp.float32),
        scratch_types=[
            pltpu.VMEM((b_per_w,), jnp.int32),
            pltpu.VMEM((2, chunk, D), jnp.float32),
            pltpu.SemaphoreType.DMA,
            pltpu.SemaphoreType.DMA,
            pltpu.SemaphoreType.DMA,
            pltpu.SemaphoreType.DMA,
        ],
        compiler_params=pltpu.CompilerParams(use_tc_tiling_on_sc=False),
    )
    def k(tok_hbm, table_hbm, out_hbm, idx_v, rows_v, sg0, sg1, so0, so1):
        wid = lax.axis_index("s") * NC + lax.axis_index("c")
        w_base = wid * b_per_w
        sg = (sg0, sg1)
        so = (so0, so1)

        def gather(g, b):
            return pltpu.make_async_copy(
                table_hbm.at[idx_v.at[pl.ds(g * chunk, chunk)]],
                rows_v.at[b], sg[b])

        def store(g, b):
            return pltpu.make_async_copy(
                rows_v.at[b], out_hbm.at[pl.ds(w_base + g * chunk, chunk)],
                so[b])

        pltpu.sync_copy(tok_hbm.at[pl.ds(w_base, b_per_w)], idx_v)

        gather(0, 0).start()
        gather(1, 1).start()

        def body(o, carry):
            for b in (0, 1):
                g = 2 * o + b
                gather(g - 2, b).wait()
                store(g - 2, b).start()
                store(g - 2, b).wait()
                gather(g, b).start()
            return carry

        lax.fori_loop(1, n_chunks // 2, body, 0)

        for b in (0, 1):
            g = n_chunks - 2 + b
            gather(g, b).wait()
            store(g, b).start()
            store(g, b).wait()

    return k


_gather = _make_gather(VOCAB_SIZE, EMB_SIZE, B_FLAT, chunk=1280)


def kernel(tokens, embedding_weight):
    flat = tokens.reshape(-1).astype(jnp.int32)
    tab128 = lax.optimization_barrier(
        embedding_weight.reshape(VOCAB_SIZE * EMB_SIZE // 128, 128))
    tab = tab128.reshape(VOCAB_SIZE, EMB_SIZE)
    out = _gather(flat, tab)
    out128 = lax.optimization_barrier(
        out.reshape(B_FLAT * EMB_SIZE // 128, 128))
    return out128.reshape(tokens.shape + (EMB_SIZE,))


# speedup vs baseline: 1.8457x; 1.0181x over previous
import functools

import jax
import jax.numpy as jnp
from jax import lax
from jax.experimental import pallas as pl
from jax.experimental.pallas import tpu as pltpu
from jax.experimental.pallas import tpu_sc as plsc

VOCAB_SIZE = 1000000
EMB_SIZE = 32
BATCH = 16384
HIST_LEN = 50


def _make_gather(V, D, B, H):
    info = plsc.get_sparse_core_info()
    NC, NS = info.num_cores, info.num_subcores
    NW = NC * NS
    bw = B // NW
    nc_blk = D // 8
    nb_blk = bw // 128
    C0_mul = nb_blk
    assert H % 2 == 0
    mesh = plsc.VectorSubcoreMesh(core_axis_name="c", subcore_axis_name="s")

    @functools.partial(
        pl.kernel,
        mesh=mesh,
        out_type=jax.ShapeDtypeStruct((H, nc_blk, B // 128, 8, 128),
                                      jnp.float32),
        scratch_types=[
            pltpu.VMEM((2, bw), jnp.int32),
            pltpu.VMEM((2, bw, D), jnp.float32),
            pltpu.VMEM((2, nc_blk, nb_blk, 8, 128), jnp.float32),
            pltpu.SemaphoreType.DMA,
            pltpu.SemaphoreType.DMA,
            pltpu.SemaphoreType.DMA,
            pltpu.SemaphoreType.DMA,
            pltpu.SemaphoreType.DMA,
            pltpu.SemaphoreType.DMA,
        ],
        compiler_params=pltpu.CompilerParams(
            use_tc_tiling_on_sc=False, needs_layout_passes=False),
    )
    def k(tok_hbm, tab_hbm, out_hbm, idx_v, rows_v, t_v,
          si0, si1, sg0, sg1, so0, so1):
        wid = lax.axis_index("s") * NC + lax.axis_index("c")
        b0 = wid * bw
        C0 = wid * C0_mul
        si = (si0, si1)
        sg = (sg0, sg1)
        so = (so0, so1)
        iota = lax.iota(jnp.int32, 16)

        def idx_copy(h, b):
            return pltpu.make_async_copy(
                tok_hbm.at[h, pl.ds(b0, bw)], idx_v.at[b], si[b])

        def gather(b):
            return pltpu.make_async_copy(
                tab_hbm.at[idx_v.at[b]], rows_v.at[b], sg[b])

        def store(h, b):
            return pltpu.make_async_copy(
                t_v.at[b], out_hbm.at[h, :, pl.ds(C0, nb_blk)], so[b])

        def transpose(b):
            rows_b = rows_v.at[b]

            @plsc.parallel_loop(0, nc_blk * nb_blk * 64, unroll=8)
            def body(kk):
                R = kk >> 8
                cb = (kk >> 6) & 3
                r = (kk >> 3) & 7
                lc = kk & 7
                tok_idx = cb * 128 + lc * 16 + iota
                c_idx = jnp.full((16,), R * 8 + r, jnp.int32)
                vals = plsc.load_gather(rows_b, [tok_idx, c_idx])
                t_v[b, R, cb, r, pl.ds(lc * 16, 16)] = vals

        def step(h, b, first=False, pre_gather=True, pre_idx=True):
            b1 = 1 - b
            if pre_gather:
                idx_copy(h + 1, b1).wait()
                gather(b1).start()
            gather(b).wait()
            if not first:
                store(h - 2, b).wait()
            transpose(b)
            store(h, b).start()
            if pre_idx:
                idx_copy(h + 2, b).start()

        idx_copy(0, 0).start()
        idx_copy(1, 1).start()
        idx_copy(0, 0).wait()
        gather(0).start()
        step(0, 0, first=True)
        step(1, 1, first=True)

        def body(o, carry):
            step(2 * o, 0)
            step(2 * o + 1, 1)
            return carry

        lax.fori_loop(1, H // 2 - 1, body, 0)

        step(H - 2, 0, pre_gather=True, pre_idx=False)
        step(H - 1, 1, pre_gather=False, pre_idx=False)
        store(H - 2, 0).wait()
        store(H - 1, 1).wait()

    return k


_gather = _make_gather(VOCAB_SIZE, EMB_SIZE, BATCH, HIST_LEN)


def kernel(tokens, embedding_weight):
    tok_t = tokens.T.astype(jnp.int32)
    tab128 = lax.optimization_barrier(
        embedding_weight.reshape(VOCAB_SIZE * EMB_SIZE // 128, 128))
    tab = tab128.reshape(VOCAB_SIZE, EMB_SIZE)
    out5 = _gather(tok_t, tab)
    out = out5.transpose(2, 4, 0, 1, 3).reshape(BATCH, HIST_LEN, EMB_SIZE)
    return out


# speedup vs baseline: 1.8783x; 1.0177x over previous
import functools

import jax
import jax.numpy as jnp
from jax import lax
from jax.experimental import pallas as pl
from jax.experimental.pallas import tpu as pltpu
from jax.experimental.pallas import tpu_sc as plsc

VOCAB_SIZE = 1000000
EMB_SIZE = 32
BATCH = 16384
HIST_LEN = 50


def _make_gather(V, D, B, H):
    info = plsc.get_sparse_core_info()
    NC, NS = info.num_cores, info.num_subcores
    NW = NC * NS
    bw = B // NW
    nc_blk = D // 8
    nb_blk = bw // 128
    C0_mul = nb_blk
    assert H % 2 == 0
    mesh = plsc.VectorSubcoreMesh(core_axis_name="c", subcore_axis_name="s")

    @functools.partial(
        pl.kernel,
        mesh=mesh,
        out_type=jax.ShapeDtypeStruct((H, nc_blk, B // 128, 8, 128),
                                      jnp.float32),
        scratch_types=[
            pltpu.VMEM((2, bw), jnp.int32),
            pltpu.VMEM((2, bw, D), jnp.float32),
            pltpu.VMEM((2, nc_blk, nb_blk, 8, 128), jnp.float32),
            pltpu.SemaphoreType.DMA,
            pltpu.SemaphoreType.DMA,
            pltpu.SemaphoreType.DMA,
            pltpu.SemaphoreType.DMA,
            pltpu.SemaphoreType.DMA,
            pltpu.SemaphoreType.DMA,
        ],
        compiler_params=pltpu.CompilerParams(
            use_tc_tiling_on_sc=False, needs_layout_passes=False),
    )
    def k(tok_hbm, tab_hbm, out_hbm, idx_v, rows_v, t_v,
          si0, si1, sg0, sg1, so0, so1):
        wid = lax.axis_index("s") * NC + lax.axis_index("c")
        b0 = wid * bw
        C0 = wid * C0_mul
        si = (si0, si1)
        sg = (sg0, sg1)
        so = (so0, so1)
        iota = lax.iota(jnp.int32, 16)

        def idx_copy(h, b):
            return pltpu.make_async_copy(
                tok_hbm.at[h, pl.ds(b0, bw)], idx_v.at[b], si[b])

        def gather(b):
            return pltpu.make_async_copy(
                tab_hbm.at[idx_v.at[b]], rows_v.at[b], sg[b])

        def store(h, b):
            return pltpu.make_async_copy(
                t_v.at[b], out_hbm.at[h, :, pl.ds(C0, nb_blk)], so[b])

        def transpose(b):
            rows_b = rows_v.at[b]
            for R in range(nc_blk):
                t_R = t_v.at[b, R]

                @plsc.parallel_loop(0, nb_blk * 64, unroll=8)
                def body(kk, _R=R, _t=t_R):
                    cb = kk >> 6
                    r = (kk >> 3) & 7
                    lc = kk & 7
                    tok_idx = cb * 128 + lc * 16 + iota
                    c_idx = jnp.full((16,), _R * 8 + r, jnp.int32)
                    vals = plsc.load_gather(rows_b, [tok_idx, c_idx])
                    _t[cb, r, pl.ds(lc * 16, 16)] = vals

        def step(h, b, first=False, pre_gather=True, pre_idx=True):
            b1 = 1 - b
            if pre_gather:
                idx_copy(h + 1, b1).wait()
                gather(b1).start()
            gather(b).wait()
            if not first:
                store(h - 2, b).wait()
            transpose(b)
            store(h, b).start()
            if pre_idx:
                idx_copy(h + 2, b).start()

        idx_copy(0, 0).start()
        idx_copy(1, 1).start()
        idx_copy(0, 0).wait()
        gather(0).start()
        step(0, 0, first=True)
        step(1, 1, first=True)

        def body(o, carry):
            step(2 * o, 0)
            step(2 * o + 1, 1)
            return carry

        lax.fori_loop(1, H // 2 - 1, body, 0)

        step(H - 2, 0, pre_gather=True, pre_idx=False)
        step(H - 1, 1, pre_gather=False, pre_idx=False)
        store(H - 2, 0).wait()
        store(H - 1, 1).wait()

    return k


_gather = _make_gather(VOCAB_SIZE, EMB_SIZE, BATCH, HIST_LEN)


def kernel(tokens, embedding_weight):
    tok_t = tokens.T.astype(jnp.int32)
    tab128 = lax.optimization_barrier(
        embedding_weight.reshape(VOCAB_SIZE * EMB_SIZE // 128, 128))
    tab = tab128.reshape(VOCAB_SIZE, EMB_SIZE)
    out5 = _gather(tok_t, tab)
    out = out5.transpose(2, 4, 0, 1, 3).reshape(BATCH, HIST_LEN, EMB_SIZE)
    return out
